# Initial kernel scaffold; baseline (speedup 1.0000x reference)
import functools

import jax
import jax.numpy as jnp
from jax import lax
from jax.experimental import pallas as pl
from jax.experimental.pallas import tpu as pltpu
from jax.experimental.pallas import tpu_sc as plsc

_N = 512
_B = 8
_K = 10
_L = 16


def _scores_body(a_ref, p_ref, sa_ref, sp_ref):
    r = lax.broadcasted_iota(jnp.int32, (_N, _N), 0)
    c = lax.broadcasted_iota(jnp.int32, (_N, _N), 1)
    ident = (r == c).astype(jnp.float32)

    def scores(A):
        At = lax.dot_general(A, ident, (((0,), (0,)), ((), ())),
                             preferred_element_type=jnp.float32)
        M = jnp.maximum(A, At) * (1.0 / _N)
        P2 = jnp.dot(M, M, preferred_element_type=jnp.float32)
        P4 = jnp.dot(P2, P2, preferred_element_type=jnp.float32)
        v = jnp.ones((8, _N), jnp.float32)
        for _ in range(8):
            v = jnp.dot(v, P4, preferred_element_type=jnp.float32)
            v = v / jnp.maximum(jnp.max(v, axis=1, keepdims=True), 1e-30)
        return v[0:1, :]

    sa_ref[...] = scores(a_ref[0])
    sp_ref[...] = scores(p_ref[0])


def _tc_scores(adj, prev_adj):
    return pl.pallas_call(
        _scores_body,
        grid=(_B,),
        in_specs=[
            pl.BlockSpec((1, _N, _N), lambda b: (b, 0, 0)),
            pl.BlockSpec((1, _N, _N), lambda b: (b, 0, 0)),
        ],
        out_specs=[
            pl.BlockSpec((1, _N), lambda b: (b, 0)),
            pl.BlockSpec((1, _N), lambda b: (b, 0)),
        ],
        out_shape=[
            jax.ShapeDtypeStruct((_B, _N), jnp.float32),
            jax.ShapeDtypeStruct((_B, _N), jnp.float32),
        ],
    )(adj, prev_adj)


def _topk16(s_ref):
    v = s_ref[pl.ds(0, _L)]
    i = lax.iota(jnp.int32, _L)
    v, i = plsc.sort_key_val(v, i, descending=True)
    for cidx in range(1, _N // _L):
        cv = s_ref[pl.ds(cidx * _L, _L)]
        ci = lax.iota(jnp.int32, _L) + cidx * _L
        cv, ci = plsc.sort_key_val(cv, ci, descending=True)
        rv = lax.rev(cv, (0,))
        ri = lax.rev(ci, (0,))
        take = v >= rv
        mv = jnp.where(take, v, rv)
        mi = jnp.where(take, i, ri)
        v, i = plsc.sort_key_val(mv, mi, descending=True)
    return i


def _gather_sub(mat_hbm, idx, base, idx_v, rows_v, sem):
    idx_v[...] = idx + base
    pltpu.async_copy(mat_hbm.at[idx_v], rows_v, sem).wait()
    rows = []
    for rr in range(_K):
        row_ids = jnp.full((_L,), rr, jnp.int32)
        rows.append(plsc.load_gather(rows_v, [row_ids, idx]))
    return rows


def _sc_body(adj_hbm, prev_hbm, sa_hbm, sp_hbm, out_hbm,
             s_v, idx_v, rows_v, shared, obuf, sem):
    core = lax.axis_index("c")
    sub = lax.axis_index("s")
    work = jnp.logical_and(core == 0, sub < _B)

    @pl.when(work)
    def _():
        t = sub
        lane = lax.iota(jnp.int32, _L)

        pltpu.sync_copy(sa_hbm.at[t], s_v)
        ia = _topk16(s_v)
        rows_a = _gather_sub(adj_hbm, ia, t * _N, idx_v, rows_v, sem)

        pltpu.sync_copy(sp_hbm.at[t], s_v)
        ip = _topk16(s_v)
        rows_p = _gather_sub(prev_hbm, ip, t * _N, idx_v, rows_v, sem)

        acc = jnp.zeros((_L,), jnp.float32)
        for rr in range(_K):
            d = rows_a[rr] - rows_p[rr]
            acc = acc + jnp.where(lane < _K, d * d, 0.0)
        total = jnp.sum(acc) * (1.0 / (_K * _K * _B))
        obuf[...] = jnp.full((_L,), total, jnp.float32)
        pltpu.sync_copy(obuf, shared.at[t])

    plsc.subcore_barrier()

    @pl.when(jnp.logical_and(core == 0, sub == 0))
    def _():
        acc = jnp.zeros((_L,), jnp.float32)
        for k in range(_B):
            pltpu.sync_copy(shared.at[k], obuf)
            acc = acc + obuf[...]
        obuf[...] = acc
        pltpu.sync_copy(obuf, out_hbm)


def _sc_loss(adj2d, prev2d, sa, sp):
    mesh = plsc.VectorSubcoreMesh(core_axis_name="c", subcore_axis_name="s",
                                  num_cores=2, num_subcores=16)
    kern = pl.kernel(
        _sc_body,
        out_type=jax.ShapeDtypeStruct((_L,), jnp.float32),
        mesh=mesh,
        scratch_types=[
            pltpu.VMEM((_N,), jnp.float32),
            pltpu.VMEM((_L,), jnp.int32),
            pltpu.VMEM((_L, _N), jnp.float32),
            pltpu.VMEM_SHARED((_B, _L), jnp.float32),
            pltpu.VMEM((_L,), jnp.float32),
            pltpu.SemaphoreType.DMA,
        ],
    )
    return kern(adj2d, prev2d, sa, sp)


def kernel(adj, prev_adj):
    sa, sp = _tc_scores(adj, prev_adj)
    adj2d = adj.reshape(_B * _N, _N)
    prev2d = prev_adj.reshape(_B * _N, _N)
    out = _sc_loss(adj2d, prev2d, sa, sp)
    return out[0]


# speedup vs baseline: 702.4303x; 702.4303x over previous
import jax
import jax.numpy as jnp
from jax import lax
from jax.experimental import pallas as pl
from jax.experimental.pallas import tpu as pltpu
from jax.experimental.pallas import tpu_sc as plsc

_N = 512
_B = 8
_K = 10
_L = 16


def _scores_body(a_ref, p_ref, sa_ref, sp_ref):
    hi = lax.Precision.HIGHEST

    def p2(A):
        Ab = A.astype(jnp.bfloat16)
        Mb = jnp.maximum(Ab, Ab.T)
        return jnp.dot(Mb, Mb,
                       preferred_element_type=jnp.float32) * (1.0 / (_N * _N))

    def step(v, M):
        v = jnp.dot(v, M, precision=hi, preferred_element_type=jnp.float32)
        return v / jnp.maximum(jnp.max(v, axis=1, keepdims=True), 1e-30)

    Pa = p2(a_ref[0])
    Pp = p2(p_ref[0])
    va = jnp.ones((8, _N), jnp.float32)
    vp = jnp.ones((8, _N), jnp.float32)
    for _ in range(12):
        va = step(va, Pa)
        vp = step(vp, Pp)
    sa_ref[...] = va[0:1, :][None]
    sp_ref[...] = vp[0:1, :][None]


def _tc_scores(adj, prev_adj):
    return pl.pallas_call(
        _scores_body,
        grid=(_B,),
        in_specs=[
            pl.BlockSpec((1, _N, _N), lambda b: (b, 0, 0)),
            pl.BlockSpec((1, _N, _N), lambda b: (b, 0, 0)),
        ],
        out_specs=[
            pl.BlockSpec((1, 1, _N), lambda b: (b, 0, 0)),
            pl.BlockSpec((1, 1, _N), lambda b: (b, 0, 0)),
        ],
        out_shape=[
            jax.ShapeDtypeStruct((_B, 1, _N), jnp.float32),
            jax.ShapeDtypeStruct((_B, 1, _N), jnp.float32),
        ],
    )(adj, prev_adj)


def _rot(x, k, tmp_ref):
    lane = lax.iota(jnp.int32, _L)
    tmp_ref[...] = x
    return plsc.load_gather(tmp_ref, [(lane + k) & (_L - 1)])


def _xlane_max(x, tmp_ref):
    for k in (8, 4, 2, 1):
        x = jnp.maximum(x, _rot(x, k, tmp_ref))
    return x


def _xlane_min_i32(x, tmp_ref):
    for k in (8, 4, 2, 1):
        x = jnp.minimum(x, _rot(x, k, tmp_ref))
    return x


def _xlane_sum(x, tmp_ref):
    for k in (8, 4, 2, 1):
        x = x + _rot(x, k, tmp_ref)
    return x


def _topk10(s_ref, ftmp, itmp):
    lane = lax.iota(jnp.int32, _L)
    neg = jnp.float32(-3e38)
    big = jnp.int32(1 << 30)
    idx_vec = jnp.zeros((_L,), jnp.int32)
    for r in range(_K):
        def body(c, carry):
            bv, bi = carry
            cv = s_ref[pl.ds(c * _L, _L)]
            ci = lane + c * _L
            take = cv > bv
            return jnp.where(take, cv, bv), jnp.where(take, ci, bi)
        bv, bi = lax.fori_loop(1, _N // _L, body,
                               (s_ref[pl.ds(0, _L)], lane))
        m = _xlane_max(bv, ftmp)
        sel = _xlane_min_i32(jnp.where(bv == m, bi, big), itmp)
        plsc.store_scatter(s_ref, [sel], jnp.full((_L,), neg, jnp.float32),
                           mask=lane == 0)
        idx_vec = jnp.where(lane == r, sel, idx_vec)
    return idx_vec


def _gather_sub(mat_hbm, idx, base, idx_v, rows_v, sem):
    idx_v[...] = idx + base
    pltpu.async_copy(mat_hbm.at[idx_v], rows_v, sem).wait()
    rows = []
    for rr in range(_K):
        row_ids = jnp.full((_L,), rr, jnp.int32)
        rows.append(plsc.load_gather(rows_v, [row_ids, idx]))
    return rows


def _sc_body(adj_hbm, prev_hbm, sa_hbm, sp_hbm, out_hbm,
             s_v, idx_v, rows_v, obuf, ftmp, itmp, sem):
    core = lax.axis_index("c")
    sub = lax.axis_index("s")
    work = jnp.logical_and(core == 0, sub < _B)
    t = jnp.minimum(sub, _B - 1)
    lane = lax.iota(jnp.int32, _L)

    pltpu.sync_copy(sa_hbm.at[t], s_v)
    ia = _topk10(s_v, ftmp, itmp)
    rows_a = _gather_sub(adj_hbm, ia, t * _N, idx_v, rows_v, sem)

    pltpu.sync_copy(sp_hbm.at[t], s_v)
    ip = _topk10(s_v, ftmp, itmp)
    rows_p = _gather_sub(prev_hbm, ip, t * _N, idx_v, rows_v, sem)

    acc = jnp.zeros((_L,), jnp.float32)
    for rr in range(_K):
        d = rows_a[rr] - rows_p[rr]
        acc = acc + jnp.where(lane < _K, d * d, 0.0)
    total = _xlane_sum(acc, ftmp)
    obuf[...] = total * (1.0 / (_K * _K * _B))

    @pl.when(work)
    def _():
        pltpu.sync_copy(obuf, out_hbm.at[t])

    plsc.subcore_barrier()

    @pl.when(jnp.logical_and(core == 0, sub == 0))
    def _():
        acc = jnp.zeros((_L,), jnp.float32)
        for k in range(_B):
            pltpu.sync_copy(out_hbm.at[k], obuf)
            acc = acc + obuf[...]
        obuf[...] = acc
        pltpu.sync_copy(obuf, out_hbm.at[0])


def _sc_loss(adj2d, prev2d, sa, sp):
    mesh = plsc.VectorSubcoreMesh(core_axis_name="c", subcore_axis_name="s",
                                  num_cores=2, num_subcores=16)
    kern = pl.kernel(
        _sc_body,
        out_type=jax.ShapeDtypeStruct((_B, _L), jnp.float32),
        mesh=mesh,
        scratch_types=[
            pltpu.VMEM((_N,), jnp.float32),
            pltpu.VMEM((_L,), jnp.int32),
            pltpu.VMEM((_L, _N), jnp.float32),
            pltpu.VMEM((_L,), jnp.float32),
            pltpu.VMEM((_L,), jnp.float32),
            pltpu.VMEM((_L,), jnp.int32),
            pltpu.SemaphoreType.DMA,
        ],
        compiler_params=pltpu.CompilerParams(needs_layout_passes=False),
    )
    return kern(adj2d, prev2d, sa, sp)


def kernel(adj, prev_adj):
    sa, sp = _tc_scores(adj, prev_adj)
    sa = sa.reshape(_B, _N)
    sp = sp.reshape(_B, _N)
    adj2d = adj.reshape(_B * _N, _N)
    prev2d = prev_adj.reshape(_B * _N, _N)
    out = _sc_loss(adj2d, prev2d, sa, sp)
    return out[0, 0]


# speedup vs baseline: 1110.3877x; 1.5808x over previous
import jax
import jax.numpy as jnp
from jax import lax
from jax.experimental import pallas as pl
from jax.experimental.pallas import tpu as pltpu
from jax.experimental.pallas import tpu_sc as plsc

_N = 512
_B = 8
_K = 10
_L = 16


def _scores_body(a_ref, p_ref, sa_ref, sp_ref):
    hi = lax.Precision.HIGHEST

    def p2(A):
        Ab = A.astype(jnp.bfloat16)
        Mb = jnp.maximum(Ab, Ab.T)
        return jnp.dot(Mb, Mb,
                       preferred_element_type=jnp.float32) * (1.0 / (_N * _N))

    def step(v, M, prec, norm):
        v = jnp.dot(v, M, precision=prec, preferred_element_type=jnp.float32)
        if norm:
            v = v / jnp.maximum(jnp.max(v, axis=1, keepdims=True), 1e-30)
        return v

    Pa = p2(a_ref[0])
    Pp = p2(p_ref[0])
    va = jnp.ones((8, _N), jnp.float32)
    vp = jnp.ones((8, _N), jnp.float32)
    precs = [lax.Precision.DEFAULT] * 8 + [hi] * 3
    for i, p in enumerate(precs):
        norm = (i % 2 == 1) or i >= len(precs) - 2
        va = step(va, Pa, p, norm)
        vp = step(vp, Pp, p, norm)
    sa_ref[...] = va[0:1, :][None]
    sp_ref[...] = vp[0:1, :][None]


def _tc_scores(adj, prev_adj):
    return pl.pallas_call(
        _scores_body,
        grid=(_B,),
        in_specs=[
            pl.BlockSpec((1, _N, _N), lambda b: (b, 0, 0)),
            pl.BlockSpec((1, _N, _N), lambda b: (b, 0, 0)),
        ],
        out_specs=[
            pl.BlockSpec((1, 1, _N), lambda b: (b, 0, 0)),
            pl.BlockSpec((1, 1, _N), lambda b: (b, 0, 0)),
        ],
        out_shape=[
            jax.ShapeDtypeStruct((_B, 1, _N), jnp.float32),
            jax.ShapeDtypeStruct((_B, 1, _N), jnp.float32),
        ],
    )(adj, prev_adj)


def _rot(x, k, tmp_ref):
    lane = lax.iota(jnp.int32, _L)
    tmp_ref[...] = x
    return plsc.load_gather(tmp_ref, [(lane + k) & (_L - 1)])


def _xlane_max(x, tmp_ref):
    for k in (8, 4, 2, 1):
        x = jnp.maximum(x, _rot(x, k, tmp_ref))
    return x


def _xlane_min_i32(x, tmp_ref):
    for k in (8, 4, 2, 1):
        x = jnp.minimum(x, _rot(x, k, tmp_ref))
    return x


def _xlane_sum(x, tmp_ref):
    for k in (8, 4, 2, 1):
        x = x + _rot(x, k, tmp_ref)
    return x


def _topk10(s_ref, ftmp, itmp):
    lane = lax.iota(jnp.int32, _L)
    neg = jnp.float32(-3e38)
    big = jnp.int32(1 << 30)
    idx_vec = jnp.zeros((_L,), jnp.int32)
    for r in range(_K):
        def body(c, carry):
            bv, bi = carry
            cv = s_ref[pl.ds(c * _L, _L)]
            ci = lane + c * _L
            take = cv > bv
            return jnp.where(take, cv, bv), jnp.where(take, ci, bi)
        bv, bi = lax.fori_loop(1, _N // _L, body,
                               (s_ref[pl.ds(0, _L)], lane))
        m = _xlane_max(bv, ftmp)
        sel = _xlane_min_i32(jnp.where(bv == m, bi, big), itmp)
        plsc.store_scatter(s_ref, [sel], jnp.full((_L,), neg, jnp.float32),
                           mask=lane == 0)
        idx_vec = jnp.where(lane == r, sel, idx_vec)
    return idx_vec


def _gather_sub(mat_hbm, idx, base, idx_v, rows_v, sem):
    idx_v[...] = idx + base
    pltpu.async_copy(mat_hbm.at[idx_v], rows_v, sem).wait()
    rows = []
    for rr in range(_K):
        row_ids = jnp.full((_L,), rr, jnp.int32)
        rows.append(plsc.load_gather(rows_v, [row_ids, idx]))
    return rows


def _sc_body(adj_hbm, prev_hbm, sa_hbm, sp_hbm, out_hbm,
             s_v, idx_v, rows_v, obuf, ftmp, itmp, sem):
    core = lax.axis_index("c")
    sub = lax.axis_index("s")
    work = jnp.logical_and(core == 0, sub < _B)
    t = jnp.minimum(sub, _B - 1)
    lane = lax.iota(jnp.int32, _L)

    pltpu.sync_copy(sa_hbm.at[t], s_v)
    ia = _topk10(s_v, ftmp, itmp)
    rows_a = _gather_sub(adj_hbm, ia, t * _N, idx_v, rows_v, sem)

    pltpu.sync_copy(sp_hbm.at[t], s_v)
    ip = _topk10(s_v, ftmp, itmp)
    rows_p = _gather_sub(prev_hbm, ip, t * _N, idx_v, rows_v, sem)

    acc = jnp.zeros((_L,), jnp.float32)
    for rr in range(_K):
        d = rows_a[rr] - rows_p[rr]
        acc = acc + jnp.where(lane < _K, d * d, 0.0)
    total = _xlane_sum(acc, ftmp)
    obuf[...] = total * (1.0 / (_K * _K * _B))

    @pl.when(work)
    def _():
        pltpu.sync_copy(obuf, out_hbm.at[t])

    plsc.subcore_barrier()

    @pl.when(jnp.logical_and(core == 0, sub == 0))
    def _():
        acc = jnp.zeros((_L,), jnp.float32)
        for k in range(_B):
            pltpu.sync_copy(out_hbm.at[k], obuf)
            acc = acc + obuf[...]
        obuf[...] = acc
        pltpu.sync_copy(obuf, out_hbm.at[0])


def _sc_loss(adj2d, prev2d, sa, sp):
    mesh = plsc.VectorSubcoreMesh(core_axis_name="c", subcore_axis_name="s",
                                  num_cores=2, num_subcores=16)
    kern = pl.kernel(
        _sc_body,
        out_type=jax.ShapeDtypeStruct((_B, _L), jnp.float32),
        mesh=mesh,
        scratch_types=[
            pltpu.VMEM((_N,), jnp.float32),
            pltpu.VMEM((_L,), jnp.int32),
            pltpu.VMEM((_L, _N), jnp.float32),
            pltpu.VMEM((_L,), jnp.float32),
            pltpu.VMEM((_L,), jnp.float32),
            pltpu.VMEM((_L,), jnp.int32),
            pltpu.SemaphoreType.DMA,
        ],
        compiler_params=pltpu.CompilerParams(needs_layout_passes=False),
    )
    return kern(adj2d, prev2d, sa, sp)


def kernel(adj, prev_adj):
    sa, sp = _tc_scores(adj, prev_adj)
    sa = sa.reshape(_B, _N)
    sp = sp.reshape(_B, _N)
    adj2d = adj.reshape(_B * _N, _N)
    prev2d = prev_adj.reshape(_B * _N, _N)
    out = _sc_loss(adj2d, prev2d, sa, sp)
    return out[0, 0]


# speedup vs baseline: 1438.7873x; 1.2958x over previous
import jax
import jax.numpy as jnp
from jax import lax
from jax.experimental import pallas as pl
from jax.experimental.pallas import tpu as pltpu
from jax.experimental.pallas import tpu_sc as plsc

_N = 512
_B = 8
_K = 10
_L = 16


def _scores_body(a_ref, p_ref, sa_ref, sp_ref):
    hi = lax.Precision.HIGHEST

    def p2(A):
        Ab = A.astype(jnp.bfloat16)
        Mb = jnp.maximum(Ab, Ab.T)
        return jnp.dot(Mb, Mb,
                       preferred_element_type=jnp.float32) * (1.0 / (_N * _N))

    def step(v, M, prec, norm):
        v = jnp.dot(v, M, precision=prec, preferred_element_type=jnp.float32)
        if norm:
            v = v / jnp.maximum(jnp.max(v, axis=1, keepdims=True), 1e-30)
        return v

    Pa = p2(a_ref[0])
    Pp = p2(p_ref[0])
    va = jnp.ones((8, _N), jnp.float32)
    vp = jnp.ones((8, _N), jnp.float32)
    precs = [lax.Precision.DEFAULT] * 6 + [hi] * 2
    for i, p in enumerate(precs):
        norm = (i % 2 == 1) or i >= len(precs) - 2
        va = step(va, Pa, p, norm)
        vp = step(vp, Pp, p, norm)
    sa_ref[...] = va[0:1, :][None]
    sp_ref[...] = vp[0:1, :][None]


def _tc_scores(adj, prev_adj):
    return pl.pallas_call(
        _scores_body,
        grid=(_B,),
        in_specs=[
            pl.BlockSpec((1, _N, _N), lambda b: (b, 0, 0)),
            pl.BlockSpec((1, _N, _N), lambda b: (b, 0, 0)),
        ],
        out_specs=[
            pl.BlockSpec((1, 1, _N), lambda b: (b, 0, 0)),
            pl.BlockSpec((1, 1, _N), lambda b: (b, 0, 0)),
        ],
        out_shape=[
            jax.ShapeDtypeStruct((_B, 1, _N), jnp.float32),
            jax.ShapeDtypeStruct((_B, 1, _N), jnp.float32),
        ],
    )(adj, prev_adj)


def _rot(x, k, tmp_ref):
    lane = lax.iota(jnp.int32, _L)
    tmp_ref[...] = x
    return plsc.load_gather(tmp_ref, [(lane + k) & (_L - 1)])


def _xlane_max(x, tmp_ref):
    for k in (8, 4, 2, 1):
        x = jnp.maximum(x, _rot(x, k, tmp_ref))
    return x


def _xlane_min_i32(x, tmp_ref):
    for k in (8, 4, 2, 1):
        x = jnp.minimum(x, _rot(x, k, tmp_ref))
    return x


def _xlane_sum(x, tmp_ref):
    for k in (8, 4, 2, 1):
        x = x + _rot(x, k, tmp_ref)
    return x


def _topk10(s_ref, ftmp, itmp):
    lane = lax.iota(jnp.int32, _L)
    neg = jnp.float32(-3e38)
    big = jnp.int32(1 << 30)
    bv = s_ref[pl.ds(0, _L)]
    bi = lane
    for c in range(1, _N // _L):
        cv = s_ref[pl.ds(c * _L, _L)]
        ci = lane + c * _L
        take = cv > bv
        bv = jnp.where(take, cv, bv)
        bi = jnp.where(take, ci, bi)
    idx_vec = jnp.zeros((_L,), jnp.int32)
    for r in range(_K):
        m = _xlane_max(bv, ftmp)
        sel = _xlane_min_i32(jnp.where(bv == m, bi, big), itmp)
        idx_vec = jnp.where(lane == r, sel, idx_vec)
        if r == _K - 1:
            break
        plsc.store_scatter(s_ref, [sel], jnp.full((_L,), neg, jnp.float32),
                           mask=lane == 0)
        lw = sel & (_L - 1)
        e0 = lane * _L + lw
        e1 = e0 + _N // 2
        g0 = plsc.load_gather(s_ref, [e0])
        g1 = plsc.load_gather(s_ref, [e1])
        take0 = g0 >= g1
        mv = jnp.where(take0, g0, g1)
        mi = jnp.where(take0, e0, e1)
        nv = _xlane_max(mv, ftmp)
        ni = _xlane_min_i32(jnp.where(mv == nv, mi, big), itmp)
        upd = lane == lw
        bv = jnp.where(upd, nv, bv)
        bi = jnp.where(upd, ni, bi)
    return idx_vec


def _cols(rows_v, idx):
    rows = []
    for rr in range(_K):
        row_ids = jnp.full((_L,), rr, jnp.int32)
        rows.append(plsc.load_gather(rows_v, [row_ids, idx]))
    return rows


def _sc_body(adj_hbm, prev_hbm, sa_hbm, sp_hbm, out_hbm,
             sa_v, sp_v, idxa_v, idxp_v, rowsa_v, rowsp_v,
             obuf, red_v, ftmp, itmp, sema, semp):
    core = lax.axis_index("c")
    sub = lax.axis_index("s")
    work = jnp.logical_and(core == 0, sub < _B)
    t = jnp.minimum(sub, _B - 1)
    lane = lax.iota(jnp.int32, _L)

    cpa = pltpu.async_copy(sa_hbm.at[t], sa_v, sema)
    cpp = pltpu.async_copy(sp_hbm.at[t], sp_v, semp)
    cpa.wait()
    ia = _topk10(sa_v, ftmp, itmp)
    idxa_v[...] = ia + t * _N
    ga = pltpu.async_copy(adj_hbm.at[idxa_v], rowsa_v, sema)
    cpp.wait()
    ip = _topk10(sp_v, ftmp, itmp)
    idxp_v[...] = ip + t * _N
    gp = pltpu.async_copy(prev_hbm.at[idxp_v], rowsp_v, semp)
    ga.wait()
    rows_a = _cols(rowsa_v, ia)
    gp.wait()
    rows_p = _cols(rowsp_v, ip)

    acc = jnp.zeros((_L,), jnp.float32)
    for rr in range(_K):
        d = rows_a[rr] - rows_p[rr]
        acc = acc + jnp.where(lane < _K, d * d, 0.0)
    total = _xlane_sum(acc, ftmp)
    obuf[...] = total * (1.0 / (_K * _K * _B))

    @pl.when(work)
    def _():
        pltpu.sync_copy(obuf, out_hbm.at[t])

    plsc.subcore_barrier()

    @pl.when(jnp.logical_and(core == 0, sub == 0))
    def _():
        pltpu.sync_copy(out_hbm, red_v)
        acc = jnp.zeros((_L,), jnp.float32)
        for k in range(_B):
            acc = acc + red_v[k]
        obuf[...] = acc
        pltpu.sync_copy(obuf, out_hbm.at[0])


def _sc_loss(adj2d, prev2d, sa, sp):
    mesh = plsc.VectorSubcoreMesh(core_axis_name="c", subcore_axis_name="s",
                                  num_cores=2, num_subcores=16)
    kern = pl.kernel(
        _sc_body,
        out_type=jax.ShapeDtypeStruct((_B, _L), jnp.float32),
        mesh=mesh,
        scratch_types=[
            pltpu.VMEM((_N,), jnp.float32),
            pltpu.VMEM((_N,), jnp.float32),
            pltpu.VMEM((_L,), jnp.int32),
            pltpu.VMEM((_L,), jnp.int32),
            pltpu.VMEM((_L, _N), jnp.float32),
            pltpu.VMEM((_L, _N), jnp.float32),
            pltpu.VMEM((_L,), jnp.float32),
            pltpu.VMEM((_B, _L), jnp.float32),
            pltpu.VMEM((_L,), jnp.float32),
            pltpu.VMEM((_L,), jnp.int32),
            pltpu.SemaphoreType.DMA,
            pltpu.SemaphoreType.DMA,
        ],
        compiler_params=pltpu.CompilerParams(needs_layout_passes=False),
    )
    return kern(adj2d, prev2d, sa, sp)


def kernel(adj, prev_adj):
    sa, sp = _tc_scores(adj, prev_adj)
    sa = sa.reshape(_B, _N)
    sp = sp.reshape(_B, _N)
    adj2d = adj.reshape(_B * _N, _N)
    prev2d = prev_adj.reshape(_B * _N, _N)
    out = _sc_loss(adj2d, prev2d, sa, sp)
    return out[0, 0]
